# Initial kernel scaffold; baseline (speedup 1.0000x reference)
import jax
import jax.numpy as jnp
from jax.experimental import pallas as pl


def kernel(words, chars, word_table, trainable_table, char_table, conv_k, conv_b, hw1_wt, hw1_bt, hw1_wh, hw1_bh, hw2_wt, hw2_bt, hw2_wh, hw2_bh):
    raise NotImplementedError("write your pallas kernel here")


# speedup vs baseline: 1.2569x; 1.2569x over previous
import functools

import jax
import jax.numpy as jnp
from jax import lax
from jax.experimental import pallas as pl
from jax.experimental.pallas import tpu as pltpu
from jax.experimental.pallas import tpu_sc as plsc

_VOCAB = 100001
_NUM_TRAINABLE = 1001
_CHAR_VOCAB = 1301
_WORD_DIM = 300
_CHAR_DIM = 200
_K = 5
_B, _S, _C = 1024, 20, 16
_WORD_RANGE = _VOCAB - _NUM_TRAINABLE
_D = _WORD_DIM + _CHAR_DIM
_BS = _B * _S

_NW = 32
_PER_TILE = _BS // _NW
_CHUNK = 128
_NCHUNK = _PER_TILE // _CHUNK

_R = 256
_NBLK = _BS // _R
_CV_PAD = 1312
_WPAD = 384


def _sc_gather(words_flat, word_table, trainable_table):
    mesh = plsc.VectorSubcoreMesh(core_axis_name="c", subcore_axis_name="s")

    @functools.partial(
        pl.kernel,
        out_type=(
            jax.ShapeDtypeStruct((_BS, _WPAD), jnp.float32),
            jax.ShapeDtypeStruct((_BS, _WPAD), jnp.float32),
        ),
        mesh=mesh,
        scratch_types=[
            pltpu.VMEM((_PER_TILE,), jnp.int32),
            pltpu.VMEM((_PER_TILE,), jnp.int32),
            pltpu.VMEM((_CHUNK, _WPAD), jnp.float32),
            pltpu.VMEM((_CHUNK, _WPAD), jnp.float32),
            pltpu.SemaphoreType.DMA,
            pltpu.SemaphoreType.DMA,
        ],
    )
    def k(words_hbm, wt_hbm, tt_hbm, wout_hbm, tout_hbm,
          idx_v, tr_v, wbuf, tbuf, sem_w, sem_t):
        wid = lax.axis_index("s") * 2 + lax.axis_index("c")
        base = wid * _PER_TILE
        pltpu.sync_copy(words_hbm.at[pl.ds(base, _PER_TILE)], idx_v)
        for i in range(_PER_TILE // 16):
            w = idx_v[pl.ds(i * 16, 16)]
            t = jnp.minimum(jnp.maximum(w - _WORD_RANGE, 0),
                            _NUM_TRAINABLE - 1)
            tr_v[pl.ds(i * 16, 16)] = t
        for j in range(_NCHUNK):
            iw = idx_v.at[pl.ds(j * _CHUNK, _CHUNK)]
            it = tr_v.at[pl.ds(j * _CHUNK, _CHUNK)]
            cw = pltpu.async_copy(wt_hbm.at[iw], wbuf, sem_w)
            ct = pltpu.async_copy(tt_hbm.at[it], tbuf, sem_t)
            cw.wait()
            ct.wait()
            pltpu.sync_copy(wbuf, wout_hbm.at[pl.ds(base + j * _CHUNK, _CHUNK)])
            pltpu.sync_copy(tbuf, tout_hbm.at[pl.ds(base + j * _CHUNK, _CHUNK)])

    return k(words_flat, word_table, trainable_table)


def _tc_body(wrows, trows, wcol, chars, ct, ck, cb,
             wt1, bt1, wh1, bh1, wt2, bt2, wh2, bh2, out):
    f32 = jnp.float32

    def mm(a, b):
        return lax.dot_general(a, b, (((1,), (0,)), ((), ())),
                               preferred_element_type=f32)

    mask = (wcol[...] > _WORD_RANGE).astype(f32)
    wv = wrows[...][:, :_WORD_DIM]
    tv = trows[...][:, :_WORD_DIM]
    wr = wv + jnp.maximum(tv, 0.0) * mask

    chars_blk = chars[...]
    table = ct[...]
    emb = []
    for c in range(_C):
        col = chars_blk[:, c:c + 1]
        oh = (col == lax.broadcasted_iota(jnp.int32, (_R, _CV_PAD), 1))
        emb.append(mm(oh.astype(jnp.bfloat16), table).astype(jnp.bfloat16))

    ckv = ck[...]
    pooled = None
    for c in range(_C):
        acc = None
        for k in range(_K):
            cc = c + k - 2
            if 0 <= cc < _C:
                y = mm(emb[cc], ckv[k])
                acc = y if acc is None else acc + y
        pooled = acc if pooled is None else jnp.maximum(pooled, acc)
    pooled = jnp.maximum(pooled + cb[...], 0.0)

    x = jnp.concatenate([wr, pooled], axis=1)
    for wt, bt, wh, bh in ((wt1, bt1, wh1, bh1), (wt2, bt2, wh2, bh2)):
        xb = x.astype(jnp.bfloat16)
        t = jax.nn.sigmoid(mm(xb, wt[...]) + bt[...])
        h = jnp.maximum(mm(xb, wh[...]) + bh[...], 0.0)
        x = t * h + (1.0 - t) * x
    out[...] = x


def _tc_forward(wrows, trows, words_col, chars2d, ct_bf, ck_bf, cb2,
                hw_args):
    row = pl.BlockSpec((_R, None), lambda i: (i, 0))

    def full(shape):
        return pl.BlockSpec(shape, lambda i: tuple(0 for _ in shape))

    in_specs = [
        pl.BlockSpec((_R, _WPAD), lambda i: (i, 0)),
        pl.BlockSpec((_R, _WPAD), lambda i: (i, 0)),
        pl.BlockSpec((_R, 1), lambda i: (i, 0)),
        pl.BlockSpec((_R, _C), lambda i: (i, 0)),
        full((_CV_PAD, _CHAR_DIM)),
        full((_K, _CHAR_DIM, _CHAR_DIM)),
        full((1, _CHAR_DIM)),
    ]
    for _ in range(2):
        in_specs += [full((_D, _D)), full((1, _D)),
                     full((_D, _D)), full((1, _D))]

    return pl.pallas_call(
        _tc_body,
        grid=(_NBLK,),
        in_specs=in_specs,
        out_specs=pl.BlockSpec((_R, _D), lambda i: (i, 0)),
        out_shape=jax.ShapeDtypeStruct((_BS, _D), jnp.float32),
        compiler_params=pltpu.CompilerParams(
            dimension_semantics=("arbitrary",)),
    )(wrows, trows, words_col, chars2d, ct_bf, ck_bf, cb2, *hw_args)


def kernel(words, chars, word_table, trainable_table, char_table, conv_k,
           conv_b, hw1_wt, hw1_bt, hw1_wh, hw1_bh, hw2_wt, hw2_bt, hw2_wh,
           hw2_bh):
    words_flat = words.reshape(_BS)
    wt_pad = jnp.pad(word_table, ((0, 0), (0, _WPAD - _WORD_DIM)))
    tt_pad = jnp.pad(trainable_table, ((0, 0), (0, _WPAD - _WORD_DIM)))
    wrows, trows = _sc_gather(words_flat, wt_pad, tt_pad)

    ct_bf = jnp.zeros((_CV_PAD, _CHAR_DIM), jnp.bfloat16)
    ct_bf = ct_bf.at[:_CHAR_VOCAB].set(char_table.astype(jnp.bfloat16))
    ck_bf = conv_k.astype(jnp.bfloat16)
    cb2 = conv_b.reshape(1, _CHAR_DIM)
    hw_args = [a.astype(jnp.bfloat16) if a.ndim == 2 else a.reshape(1, _D)
               for a in (hw1_wt, hw1_bt, hw1_wh, hw1_bh,
                         hw2_wt, hw2_bt, hw2_wh, hw2_bh)]

    out = _tc_forward(wrows, trows, words_flat.reshape(_BS, 1),
                      chars.reshape(_BS, _C), ct_bf, ck_bf, cb2, hw_args)
    return out.reshape(_B, _S, _D)
